# Initial kernel scaffold; baseline (speedup 1.0000x reference)
import jax
import jax.numpy as jnp
from jax.experimental import pallas as pl


def kernel(features, edge_index, edge_weight, W1, b1, W2, b2, W3, b3):
    raise NotImplementedError("write your pallas kernel here")


# speedup vs baseline: 4.6208x; 4.6208x over previous
import functools

import jax
import jax.numpy as jnp
from jax import lax
from jax.experimental import pallas as pl
from jax.experimental.pallas import tpu as pltpu
from jax.experimental.pallas import tpu_sc as plsc

DIM = 128
NC, NS = 2, 16
NW = NC * NS
CHUNK = 128
LANES = 16


@functools.lru_cache(maxsize=None)
def _make_spmm(n_nodes: int, e_pad: int):
    epw = e_pad // NW
    nch = epw // CHUNK
    rows_per_sub = n_nodes // NS

    mesh = plsc.VectorSubcoreMesh(core_axis_name="c", subcore_axis_name="s")

    def body(h_hbm, src_hbm, dst_hbm, w_hbm, out_hbm,
             idx_s, idx_d, w_v, rows, agg, sem):
        c = lax.axis_index("c")
        s = lax.axis_index("s")
        wid = c * NS + s

        def zrow(r, _):
            for f in range(DIM // LANES):
                rows[r, pl.ds(f * LANES, LANES)] = jnp.zeros((LANES,),
                                                             jnp.float32)
            return 0
        lax.fori_loop(0, CHUNK, zrow, 0)
        full, rem = divmod(rows_per_sub, CHUNK)
        for j in range(full):
            pltpu.sync_copy(rows, agg.at[pl.ds(s * rows_per_sub + j * CHUNK,
                                               CHUNK)])
        if rem:
            pltpu.sync_copy(rows.at[pl.ds(0, rem)],
                            agg.at[pl.ds(s * rows_per_sub + full * CHUNK,
                                         rem)])
        plsc.subcore_barrier()

        def chunk_body(i, _):
            base = wid * epw + i * CHUNK
            pltpu.sync_copy(src_hbm.at[pl.ds(base, CHUNK)], idx_s)
            pltpu.sync_copy(dst_hbm.at[pl.ds(base, CHUNK)], idx_d)
            pltpu.sync_copy(w_hbm.at[pl.ds(base, CHUNK)], w_v)
            pltpu.async_copy(h_hbm.at[idx_s], rows, sem).wait()

            def scale(j, _):
                e0 = j * LANES
                wvec = w_v[pl.ds(e0, LANES)]
                for k in range(LANES):
                    wk = jnp.full((LANES,), wvec[k], jnp.float32)
                    for f in range(DIM // LANES):
                        sl = pl.ds(f * LANES, LANES)
                        rows[e0 + k, sl] = rows[e0 + k, sl] * wk
                return 0
            lax.fori_loop(0, CHUNK // LANES, scale, 0)

            pltpu.sync_copy(rows, agg.at[idx_d], add=True)
            return 0
        lax.fori_loop(0, nch, chunk_body, 0)

        plsc.subcore_barrier()
        pltpu.sync_copy(agg.at[pl.ds(s * rows_per_sub, rows_per_sub)],
                        out_hbm.at[c, pl.ds(s * rows_per_sub, rows_per_sub)])

    return pl.kernel(
        body,
        out_type=jax.ShapeDtypeStruct((NC, n_nodes, DIM), jnp.float32),
        mesh=mesh,
        scratch_types=[
            pltpu.VMEM((CHUNK,), jnp.int32),
            pltpu.VMEM((CHUNK,), jnp.int32),
            pltpu.VMEM((CHUNK,), jnp.float32),
            pltpu.VMEM((CHUNK, DIM), jnp.float32),
            pltpu.VMEM_SHARED((n_nodes, DIM), jnp.float32),
            pltpu.SemaphoreType.DMA,
        ],
    )


@functools.lru_cache(maxsize=None)
def _make_tc(n_nodes: int, do_relu: bool):
    block_rows = n_nodes // 8
    assert n_nodes % 8 == 0 and block_rows % 8 == 0
    grid = (n_nodes // block_rows,)

    def body(p_ref, w_ref, b_ref, o_ref):
        x = p_ref[0] + p_ref[1]
        y = jnp.dot(x, w_ref[...], preferred_element_type=jnp.float32,
                    precision=lax.Precision.HIGHEST)
        y = y + b_ref[...]
        if do_relu:
            y = jnp.maximum(y, 0.0)
        o_ref[...] = y

    return pl.pallas_call(
        body,
        grid=grid,
        in_specs=[
            pl.BlockSpec((NC, block_rows, DIM), lambda i: (0, i, 0)),
            pl.BlockSpec((DIM, DIM), lambda i: (0, 0)),
            pl.BlockSpec((1, DIM), lambda i: (0, 0)),
        ],
        out_specs=pl.BlockSpec((block_rows, DIM), lambda i: (i, 0)),
        out_shape=jax.ShapeDtypeStruct((n_nodes, DIM), jnp.float32),
    )


def kernel(features, edge_index, edge_weight, W1, b1, W2, b2, W3, b3):
    n_nodes = features.shape[0]
    e = edge_index.shape[1]
    unit = NW * CHUNK
    e_pad = ((e + unit - 1) // unit) * unit
    pad = e_pad - e
    row_unit = NS * 128
    n_pad = ((n_nodes + row_unit - 1) // row_unit) * row_unit

    src = edge_index[0]
    dst = edge_index[1]
    w = edge_weight.astype(jnp.float32)
    if pad:
        pad_idx = (jnp.arange(pad, dtype=jnp.int32) % n_nodes)
        src = jnp.concatenate([src, pad_idx])
        dst = jnp.concatenate([dst, pad_idx])
        w = jnp.concatenate([w, jnp.zeros((pad,), jnp.float32)])

    spmm = _make_spmm(n_pad, e_pad)
    tc_relu = _make_tc(n_pad, True)
    tc_id = _make_tc(n_pad, False)

    x = jnp.pad(features.astype(jnp.float32), ((0, n_pad - n_nodes), (0, 0)))
    b1r = b1.reshape(1, DIM).astype(jnp.float32)
    b2r = b2.reshape(1, DIM).astype(jnp.float32)
    b3r = b3.reshape(1, DIM).astype(jnp.float32)

    p = spmm(x, src, dst, w)
    x = tc_relu(p, W1.astype(jnp.float32), b1r)
    p = spmm(x, src, dst, w)
    x = tc_relu(p, W2.astype(jnp.float32), b2r)
    p = spmm(x, src, dst, w)
    out = tc_id(p, W3.astype(jnp.float32), b3r)
    return out[:n_nodes]


# speedup vs baseline: 11.3665x; 2.4599x over previous
import functools

import jax
import jax.numpy as jnp
from jax import lax
from jax.experimental import pallas as pl
from jax.experimental.pallas import tpu as pltpu
from jax.experimental.pallas import tpu_sc as plsc

DIM = 128
NC, NS = 2, 16
NW = NC * NS
CHUNK = 112
LANES = 16
NBUF = 3
BLK = 6


@functools.lru_cache(maxsize=None)
def _make_spmm(n_nodes: int, e_pad: int):
    epw = e_pad // NW
    nch = epw // CHUNK
    assert nch % NBUF == 0 and nch % BLK == 0 and nch >= 2 * BLK
    ngrp = nch // NBUF
    nblk = nch // BLK
    rows_per_sub = n_nodes // NS

    mesh = plsc.VectorSubcoreMesh(core_axis_name="c", subcore_axis_name="s")

    def body(h_hbm, src_hbm, dst_hbm, w_hbm, out_hbm,
             src_c, dst_c, w_c,
             r0, r1, r2, agg,
             g0, g1, g2, s0, s1, s2, rsem):
        c = lax.axis_index("c")
        s = lax.axis_index("s")
        wid = c * NS + s
        rows = (r0, r1, r2)
        gsem = (g0, g1, g2)
        ssem = (s0, s1, s2)

        def zrow(r, _):
            for f in range(DIM // LANES):
                r0[r, pl.ds(f * LANES, LANES)] = jnp.zeros((LANES,),
                                                           jnp.float32)
            return 0
        lax.fori_loop(0, CHUNK, zrow, 0)
        full, rem = divmod(rows_per_sub, CHUNK)
        for j in range(full):
            pltpu.sync_copy(r0, agg.at[pl.ds(s * rows_per_sub + j * CHUNK,
                                             CHUNK)])
        if rem:
            pltpu.sync_copy(r0.at[pl.ds(0, rem)],
                            agg.at[pl.ds(s * rows_per_sub + full * CHUNK,
                                         rem)])
        plsc.subcore_barrier()

        def refill_descs(blk):
            h = lax.rem(blk, 2)
            rsl = pl.ds(h * BLK, BLK)
            return (
                (src_hbm.at[wid, blk], src_c.at[rsl]),
                (dst_hbm.at[wid, blk], dst_c.at[rsl]),
                (w_hbm.at[wid, blk], w_c.at[rsl]),
            )

        def start_refill(blk):
            for sd in refill_descs(blk):
                pltpu.async_copy(sd[0], sd[1], rsem)

        def wait_refill(blk):
            for sd in refill_descs(blk):
                pltpu.make_async_copy(sd[0], sd[1], rsem).wait()

        def scale(buf, i):
            r = lax.rem(i, 2 * BLK)

            def sbody(j, _):
                e0 = j * LANES
                wvec = w_c[r, pl.ds(e0, LANES)]
                for k in range(LANES):
                    wk = jnp.full((LANES,), wvec[k], jnp.float32)
                    for f in range(DIM // LANES):
                        sl = pl.ds(f * LANES, LANES)
                        buf[e0 + k, sl] = buf[e0 + k, sl] * wk
                return 0
            lax.fori_loop(0, CHUNK // LANES, sbody, 0)

        def start_gather(b, i):
            pltpu.async_copy(h_hbm.at[src_c.at[lax.rem(i, 2 * BLK)]],
                             rows[b], gsem[b])

        def wait_gather(b, i):
            pltpu.make_async_copy(h_hbm.at[src_c.at[lax.rem(i, 2 * BLK)]],
                                  rows[b], gsem[b]).wait()

        def start_scatter(b, i):
            pltpu.async_copy(rows[b], agg.at[dst_c.at[lax.rem(i, 2 * BLK)]],
                             ssem[b], add=True)

        def wait_scatter(b, i):
            pltpu.make_async_copy(rows[b],
                                  agg.at[dst_c.at[lax.rem(i, 2 * BLK)]],
                                  ssem[b]).wait()

        for sd in refill_descs(0):
            pltpu.sync_copy(sd[0], sd[1])
        start_gather(0, 0)
        start_gather(1, 1)

        def grp(g, _):
            for b in range(NBUF):
                i = g * NBUF + b
                b2 = (b + 2) % NBUF
                wait_gather(b, i)
                scale(rows[b], i)

                @pl.when(i >= 1)
                def _():
                    wait_scatter(b2, i - 1)

                @pl.when((lax.rem(i + 2, BLK) == 0) & (i + 2 < nch))
                def _():
                    wait_refill((i + 2) // BLK)

                @pl.when(i + 2 < nch)
                def _():
                    start_gather(b2, i + 2)

                @pl.when((lax.rem(i, BLK) == 0) & (i + BLK < nch))
                def _():
                    start_refill(i // BLK + 1)

                start_scatter(b, i)
            return 0
        lax.fori_loop(0, ngrp, grp, 0)

        wait_scatter((nch - 1) % NBUF, nch - 1)

        plsc.subcore_barrier()
        pltpu.sync_copy(agg.at[pl.ds(s * rows_per_sub, rows_per_sub)],
                        out_hbm.at[c, pl.ds(s * rows_per_sub, rows_per_sub)])

    return pl.kernel(
        body,
        out_type=jax.ShapeDtypeStruct((NC, n_nodes, DIM), jnp.float32),
        mesh=mesh,
        scratch_types=[
            pltpu.VMEM((2 * BLK, CHUNK), jnp.int32),
            pltpu.VMEM((2 * BLK, CHUNK), jnp.int32),
            pltpu.VMEM((2 * BLK, CHUNK), jnp.float32),
            pltpu.VMEM((CHUNK, DIM), jnp.float32),
            pltpu.VMEM((CHUNK, DIM), jnp.float32),
            pltpu.VMEM((CHUNK, DIM), jnp.float32),
            pltpu.VMEM_SHARED((n_nodes, DIM), jnp.float32),
            pltpu.SemaphoreType.DMA,
            pltpu.SemaphoreType.DMA,
            pltpu.SemaphoreType.DMA,
            pltpu.SemaphoreType.DMA,
            pltpu.SemaphoreType.DMA,
            pltpu.SemaphoreType.DMA,
            pltpu.SemaphoreType.DMA,
        ],
    )


@functools.lru_cache(maxsize=None)
def _make_tc(n_nodes: int, do_relu: bool):
    block_rows = n_nodes // 8
    assert n_nodes % 8 == 0 and block_rows % 8 == 0
    grid = (n_nodes // block_rows,)

    def body(p_ref, w_ref, b_ref, o_ref):
        x = p_ref[0] + p_ref[1]
        y = jnp.dot(x, w_ref[...], preferred_element_type=jnp.float32,
                    precision=lax.Precision.HIGHEST)
        y = y + b_ref[...]
        if do_relu:
            y = jnp.maximum(y, 0.0)
        o_ref[...] = y

    return pl.pallas_call(
        body,
        grid=grid,
        in_specs=[
            pl.BlockSpec((NC, block_rows, DIM), lambda i: (0, i, 0)),
            pl.BlockSpec((DIM, DIM), lambda i: (0, 0)),
            pl.BlockSpec((1, DIM), lambda i: (0, 0)),
        ],
        out_specs=pl.BlockSpec((block_rows, DIM), lambda i: (i, 0)),
        out_shape=jax.ShapeDtypeStruct((n_nodes, DIM), jnp.float32),
    )


def kernel(features, edge_index, edge_weight, W1, b1, W2, b2, W3, b3):
    n_nodes = features.shape[0]
    e = edge_index.shape[1]
    unit = NW * CHUNK * NBUF * BLK
    e_pad = ((e + unit - 1) // unit) * unit
    pad = e_pad - e
    row_unit = NS * 128
    n_pad = ((n_nodes + row_unit - 1) // row_unit) * row_unit

    src = edge_index[0]
    dst = edge_index[1]
    w = edge_weight.astype(jnp.float32)
    if pad:
        pad_idx = (jnp.arange(pad, dtype=jnp.int32) % n_nodes)
        src = jnp.concatenate([src, pad_idx])
        dst = jnp.concatenate([dst, pad_idx])
        w = jnp.concatenate([w, jnp.zeros((pad,), jnp.float32)])

    nch = e_pad // NW // CHUNK
    nblk = nch // BLK
    src4 = src.reshape(NW, nblk, BLK, CHUNK)
    dst4 = dst.reshape(NW, nblk, BLK, CHUNK)
    w4 = w.reshape(NW, nblk, BLK, CHUNK)

    spmm = _make_spmm(n_pad, e_pad)
    tc_relu = _make_tc(n_pad, True)
    tc_id = _make_tc(n_pad, False)

    x = jnp.pad(features.astype(jnp.float32), ((0, n_pad - n_nodes), (0, 0)))
    b1r = b1.reshape(1, DIM).astype(jnp.float32)
    b2r = b2.reshape(1, DIM).astype(jnp.float32)
    b3r = b3.reshape(1, DIM).astype(jnp.float32)

    p = spmm(x, src4, dst4, w4)
    x = tc_relu(p, W1.astype(jnp.float32), b1r)
    p = spmm(x, src4, dst4, w4)
    x = tc_relu(p, W2.astype(jnp.float32), b2r)
    p = spmm(x, src4, dst4, w4)
    out = tc_id(p, W3.astype(jnp.float32), b3r)
    return out[:n_nodes]


# speedup vs baseline: 11.4703x; 1.0091x over previous
import functools

import jax
import jax.numpy as jnp
from jax import lax
from jax.experimental import pallas as pl
from jax.experimental.pallas import tpu as pltpu
from jax.experimental.pallas import tpu_sc as plsc

DIM = 128
NC, NS = 2, 16
NW = NC * NS
CHUNK = 112
LANES = 16
NBUF = 3
BLK = 6


@functools.lru_cache(maxsize=None)
def _make_spmm(n_nodes: int, e_pad: int):
    epw = e_pad // NW
    nch = epw // CHUNK
    assert nch % NBUF == 0 and nch % BLK == 0 and nch >= 2 * BLK
    ngrp = nch // NBUF
    nblk = nch // BLK
    rows_per_sub = n_nodes // NS

    mesh = plsc.VectorSubcoreMesh(core_axis_name="c", subcore_axis_name="s")

    def body(h_hbm, src_hbm, dst_hbm, w_hbm, out_hbm,
             src_c, dst_c, w_c,
             r0, r1, r2, agg,
             g0, g1, g2, s0, s1, s2, rsem):
        c = lax.axis_index("c")
        s = lax.axis_index("s")
        wid = c * NS + s
        rows = (r0, r1, r2)
        gsem = (g0, g1, g2)
        ssem = (s0, s1, s2)

        def zrow(r, _):
            for f in range(DIM // LANES):
                r0[r, pl.ds(f * LANES, LANES)] = jnp.zeros((LANES,),
                                                           jnp.float32)
            return 0
        lax.fori_loop(0, CHUNK, zrow, 0)
        full, rem = divmod(rows_per_sub, CHUNK)
        for j in range(full):
            pltpu.sync_copy(r0, agg.at[pl.ds(s * rows_per_sub + j * CHUNK,
                                             CHUNK)])
        if rem:
            pltpu.sync_copy(r0.at[pl.ds(0, rem)],
                            agg.at[pl.ds(s * rows_per_sub + full * CHUNK,
                                         rem)])
        plsc.subcore_barrier()

        def refill_descs(blk):
            h = lax.rem(blk, 2)
            rsl = pl.ds(h * BLK, BLK)
            return (
                (src_hbm.at[wid, blk], src_c.at[rsl]),
                (dst_hbm.at[wid, blk], dst_c.at[rsl]),
                (w_hbm.at[wid, blk], w_c.at[rsl]),
            )

        def start_refill(blk):
            for sd in refill_descs(blk):
                pltpu.async_copy(sd[0], sd[1], rsem)

        def wait_refill(blk):
            for sd in refill_descs(blk):
                pltpu.make_async_copy(sd[0], sd[1], rsem).wait()

        def scale(buf, i):
            r = lax.rem(i, 2 * BLK)

            def sbody(j, _):
                e0 = j * LANES
                wvec = w_c[r, pl.ds(e0, LANES)]
                for k in range(LANES):
                    wk = jnp.full((LANES,), wvec[k], jnp.float32)
                    for f in range(DIM // LANES):
                        sl = pl.ds(f * LANES, LANES)
                        buf[e0 + k, sl] = buf[e0 + k, sl] * wk
                return 0
            lax.fori_loop(0, CHUNK // LANES, sbody, 0)

        def start_gather(b, i):
            pltpu.async_copy(h_hbm.at[src_c.at[lax.rem(i, 2 * BLK)]],
                             rows[b], gsem[b])

        def wait_gather(b, i):
            pltpu.make_async_copy(h_hbm.at[src_c.at[lax.rem(i, 2 * BLK)]],
                                  rows[b], gsem[b]).wait()

        def start_scatter(b, i):
            pltpu.async_copy(rows[b], agg.at[dst_c.at[lax.rem(i, 2 * BLK)]],
                             ssem[b], add=True)

        def wait_scatter(b, i):
            pltpu.make_async_copy(rows[b],
                                  agg.at[dst_c.at[lax.rem(i, 2 * BLK)]],
                                  ssem[b]).wait()

        for sd in refill_descs(0):
            pltpu.sync_copy(sd[0], sd[1])
        start_gather(0, 0)
        start_gather(1, 1)

        def grp(g, _):
            for b in range(NBUF):
                i = g * NBUF + b
                b2 = (b + 2) % NBUF
                wait_gather(b, i)
                scale(rows[b], i)

                @pl.when(i >= 1)
                def _():
                    wait_scatter(b2, i - 1)

                @pl.when((lax.rem(i + 2, BLK) == 0) & (i + 2 < nch))
                def _():
                    wait_refill((i + 2) // BLK)

                @pl.when(i + 2 < nch)
                def _():
                    start_gather(b2, i + 2)

                @pl.when((lax.rem(i, BLK) == 0) & (i + BLK < nch))
                def _():
                    start_refill(i // BLK + 1)

                start_scatter(b, i)
            return 0
        lax.fori_loop(0, ngrp, grp, 0)

        wait_scatter((nch - 1) % NBUF, nch - 1)

        plsc.subcore_barrier()
        pltpu.sync_copy(agg.at[pl.ds(s * rows_per_sub, rows_per_sub)],
                        out_hbm.at[c, pl.ds(s * rows_per_sub, rows_per_sub)])

    return pl.kernel(
        body,
        out_type=jax.ShapeDtypeStruct((NC, n_nodes, DIM), jnp.float32),
        mesh=mesh,
        scratch_types=[
            pltpu.VMEM((2 * BLK, CHUNK), jnp.int32),
            pltpu.VMEM((2 * BLK, CHUNK), jnp.int32),
            pltpu.VMEM((2 * BLK, CHUNK), jnp.float32),
            pltpu.VMEM((CHUNK, DIM), jnp.float32),
            pltpu.VMEM((CHUNK, DIM), jnp.float32),
            pltpu.VMEM((CHUNK, DIM), jnp.float32),
            pltpu.VMEM_SHARED((n_nodes, DIM), jnp.float32),
            pltpu.SemaphoreType.DMA,
            pltpu.SemaphoreType.DMA,
            pltpu.SemaphoreType.DMA,
            pltpu.SemaphoreType.DMA,
            pltpu.SemaphoreType.DMA,
            pltpu.SemaphoreType.DMA,
            pltpu.SemaphoreType.DMA,
        ],
    )


@functools.lru_cache(maxsize=None)
def _make_tc(n_nodes: int, do_relu: bool):
    block_rows = n_nodes // 8
    assert n_nodes % 8 == 0 and block_rows % 8 == 0
    grid = (n_nodes // block_rows,)

    def body(p_ref, w_ref, b_ref, o_ref):
        x = p_ref[0] + p_ref[1]
        w = w_ref[...]
        xh = x.astype(jnp.bfloat16)
        xl = (x - xh.astype(jnp.float32)).astype(jnp.bfloat16)
        wh = w.astype(jnp.bfloat16)
        wl = (w - wh.astype(jnp.float32)).astype(jnp.bfloat16)

        def mm(a, c):
            return jnp.dot(a, c, preferred_element_type=jnp.float32)

        y = mm(xh, wl) + mm(xl, wh)
        y = y + mm(xh, wh)
        y = y + b_ref[...]
        if do_relu:
            y = jnp.maximum(y, 0.0)
        o_ref[...] = y

    return pl.pallas_call(
        body,
        grid=grid,
        in_specs=[
            pl.BlockSpec((NC, block_rows, DIM), lambda i: (0, i, 0)),
            pl.BlockSpec((DIM, DIM), lambda i: (0, 0)),
            pl.BlockSpec((1, DIM), lambda i: (0, 0)),
        ],
        out_specs=pl.BlockSpec((block_rows, DIM), lambda i: (i, 0)),
        out_shape=jax.ShapeDtypeStruct((n_nodes, DIM), jnp.float32),
    )


def kernel(features, edge_index, edge_weight, W1, b1, W2, b2, W3, b3):
    n_nodes = features.shape[0]
    e = edge_index.shape[1]
    unit = NW * CHUNK * NBUF * BLK
    e_pad = ((e + unit - 1) // unit) * unit
    pad = e_pad - e
    row_unit = NS * 128
    n_pad = ((n_nodes + row_unit - 1) // row_unit) * row_unit

    src = edge_index[0]
    dst = edge_index[1]
    w = edge_weight.astype(jnp.float32)
    if pad:
        pad_idx = (jnp.arange(pad, dtype=jnp.int32) % n_nodes)
        src = jnp.concatenate([src, pad_idx])
        dst = jnp.concatenate([dst, pad_idx])
        w = jnp.concatenate([w, jnp.zeros((pad,), jnp.float32)])

    nch = e_pad // NW // CHUNK
    nblk = nch // BLK
    src4 = src.reshape(NW, nblk, BLK, CHUNK)
    dst4 = dst.reshape(NW, nblk, BLK, CHUNK)
    w4 = w.reshape(NW, nblk, BLK, CHUNK)

    spmm = _make_spmm(n_pad, e_pad)
    tc_relu = _make_tc(n_pad, True)
    tc_id = _make_tc(n_pad, False)

    x = jnp.pad(features.astype(jnp.float32), ((0, n_pad - n_nodes), (0, 0)))
    b1r = b1.reshape(1, DIM).astype(jnp.float32)
    b2r = b2.reshape(1, DIM).astype(jnp.float32)
    b3r = b3.reshape(1, DIM).astype(jnp.float32)

    p = spmm(x, src4, dst4, w4)
    x = tc_relu(p, W1.astype(jnp.float32), b1r)
    p = spmm(x, src4, dst4, w4)
    x = tc_relu(p, W2.astype(jnp.float32), b2r)
    p = spmm(x, src4, dst4, w4)
    out = tc_id(p, W3.astype(jnp.float32), b3r)
    return out[:n_nodes]
